# Initial kernel scaffold; baseline (speedup 1.0000x reference)
import jax
import jax.numpy as jnp
from jax.experimental import pallas as pl


def kernel(step, lprobs, mask_stop_search, scores, prev_indices, original_batch_idxs):
    raise NotImplementedError("write your pallas kernel here")


# speedup vs baseline: 3.5911x; 3.5911x over previous
import functools

import jax
import jax.numpy as jnp
from jax import lax
from jax.experimental import pallas as pl
from jax.experimental.pallas import tpu as pltpu
from jax.experimental.pallas import tpu_sc as plsc

PAD = 1
V = 100000
G = 4
DIV = -0.5
BSZ = 32
BEAM = 8
ROWS = BSZ * BEAM
NC, NS, L = 2, 16, 16
NW = NC * NS
RPW = ROWS // NW
HALF = V // 2
NV = HALF // L
CAPL = 48
CAP = CAPL * L
NEG = float("-inf")


def _sc_body(lp_hbm, vals_hbm, idx_hbm, buf0, buf1, cv, ci, sem0, sem1):
    wid = lax.axis_index("s") * NC + lax.axis_index("c")
    neg16 = jnp.full((L,), NEG, jnp.float32)
    lane = lax.iota(jnp.int32, L)

    def row_body(j, _):
        r = wid * RPW + j
        base = pl.multiple_of(r * V, 8)
        cp0 = pltpu.async_copy(lp_hbm.at[pl.ds(base, HALF)], buf0, sem0)
        cp1 = pltpu.async_copy(lp_hbm.at[pl.ds(base + HALF, HALF)], buf1, sem1)

        def scanmax(buf):
            def b(i, mv):
                return jnp.maximum(mv, buf[pl.ds(i * L, L)])
            return b

        cp0.wait()
        mv = lax.fori_loop(0, NV, scanmax(buf0), neg16)
        cp1.wait()
        mv = lax.fori_loop(0, NV, scanmax(buf1), mv)
        t = mv[0]
        for q in range(1, L):
            t = jnp.minimum(t, mv[q])
        tvec = jnp.full((L,), t, jnp.float32)

        def clr(i, _):
            cv[pl.ds(i * L, L)] = neg16
            ci[pl.ds(i * L, L)] = jnp.zeros((L,), jnp.int32)
            return 0

        lax.fori_loop(0, CAPL, clr, 0)

        def passb(buf, off):
            def b(i, kl):
                v = buf[pl.ds(i * L, L)]
                m = v >= tvec
                iv = jnp.full((L,), off + i * L, jnp.int32) + lane
                pos = kl * L + lane
                plsc.store_scatter(cv, [pos], v, mask=m)
                plsc.store_scatter(ci, [pos], iv, mask=m)
                return jnp.minimum(kl + m.astype(jnp.int32), CAPL - 1)
            return b

        kl = lax.fori_loop(0, NV, passb(buf0, 0), jnp.zeros((L,), jnp.int32))
        lax.fori_loop(0, NV, passb(buf1, HALF), kl)

        ob = pl.multiple_of(r * CAP, 8)
        pltpu.sync_copy(cv.at[pl.ds(0, CAP)], vals_hbm.at[pl.ds(ob, CAP)])
        pltpu.sync_copy(ci.at[pl.ds(0, CAP)], idx_hbm.at[pl.ds(ob, CAP)])
        return 0

    lax.fori_loop(0, RPW, row_body, 0)


def _sc_candidates(lp_flat):
    mesh = plsc.VectorSubcoreMesh(
        core_axis_name="c", subcore_axis_name="s", num_cores=NC, num_subcores=NS
    )
    k = pl.kernel(
        _sc_body,
        out_type=[
            jax.ShapeDtypeStruct((ROWS * CAP,), jnp.float32),
            jax.ShapeDtypeStruct((ROWS * CAP,), jnp.int32),
        ],
        mesh=mesh,
        scratch_types=[
            pltpu.VMEM((HALF,), jnp.float32),
            pltpu.VMEM((HALF,), jnp.float32),
            pltpu.VMEM((CAP,), jnp.float32),
            pltpu.VMEM((CAP,), jnp.int32),
            pltpu.SemaphoreType.DMA,
            pltpu.SemaphoreType.DMA,
        ],
        compiler_params=pltpu.CompilerParams(needs_layout_passes=False),
    )
    return k(lp_flat)


def _fix_body(cvals_ref, cidx_ref, bias_ref, mask_ref, so_ref, io_ref, bo_ref):
    BIG = jnp.int32(1 << 30)
    pen_toks = []
    scols = [None] * BEAM
    icols = [None] * BEAM
    bcols = [None] * BEAM
    for g in range(G):
        v0 = cvals_ref[:, g * CAP:(g + 1) * CAP] + bias_ref[:, g:g + 1]
        v1 = cvals_ref[:, (g + 4) * CAP:(g + 5) * CAP] + bias_ref[:, g + 4:g + 5]
        i0 = cidx_ref[:, g * CAP:(g + 1) * CAP]
        i1 = cidx_ref[:, (g + 4) * CAP:(g + 5) * CAP]
        if g > 0:
            p0 = jnp.zeros_like(v0)
            p1 = jnp.zeros_like(v1)
            for tk in pen_toks:
                p0 += (i0 == tk).astype(jnp.float32)
                p1 += (i1 == tk).astype(jnp.float32)
            v0 = v0 + DIV * p0
            v1 = v1 + DIV * p1
        v = jnp.concatenate([v0, v1], axis=1)
        f = jnp.concatenate([i0, i1 + V], axis=1)
        for k in range(2):
            mx = jnp.max(v, axis=1, keepdims=True)
            fi = jnp.min(jnp.where(v == mx, f, BIG), axis=1, keepdims=True)
            bm = (fi >= V).astype(jnp.int32)
            tok = fi - bm * V
            msk = jnp.where(bm == 0, mask_ref[:, g:g + 1], mask_ref[:, g + 4:g + 5])
            tokm = jnp.where(msk == 0, PAD, tok)
            scols[k * 4 + g] = mx
            icols[k * 4 + g] = tokm
            bcols[k * 4 + g] = bm * G + g
            pen_toks.append(tokm)
            if k == 0:
                v = jnp.where(f == fi, NEG, v)
    so_ref[...] = jnp.concatenate(scols, axis=1)
    io_ref[...] = jnp.concatenate(icols, axis=1)
    bo_ref[...] = jnp.concatenate(bcols, axis=1)


def _fixup(cvals, cidx, bias, mask, interpret=False):
    return pl.pallas_call(
        _fix_body,
        out_shape=[
            jax.ShapeDtypeStruct((BSZ, BEAM), jnp.float32),
            jax.ShapeDtypeStruct((BSZ, BEAM), jnp.int32),
            jax.ShapeDtypeStruct((BSZ, BEAM), jnp.int32),
        ],
        interpret=interpret,
    )(cvals, cidx, bias, mask)


def kernel(step, lprobs, mask_stop_search, scores, prev_indices, original_batch_idxs):
    lp_flat = lprobs.reshape(ROWS * V)
    cand_vals, cand_idx = _sc_candidates(lp_flat)
    step_i = jnp.asarray(step, jnp.int32)
    sc_step = lax.dynamic_index_in_dim(scores, step_i, axis=2, keepdims=False)
    m0 = (jnp.arange(BEAM, dtype=jnp.int32) // 4) == 0
    bias = jnp.where(step_i == 0, jnp.where(m0[None, :], 0.0, NEG), sc_step)
    out = _fixup(
        cand_vals.reshape(BSZ, BEAM * CAP),
        cand_idx.reshape(BSZ, BEAM * CAP),
        bias.astype(jnp.float32),
        mask_stop_search,
    )
    return (out[0], out[1], out[2])


# speedup vs baseline: 7.2366x; 2.0152x over previous
import functools

import jax
import jax.numpy as jnp
from jax import lax
from jax.experimental import pallas as pl
from jax.experimental.pallas import tpu as pltpu
from jax.experimental.pallas import tpu_sc as plsc

PAD = 1
V = 100000
G = 4
DIV = -0.5
BSZ = 32
BEAM = 8
ROWS = BSZ * BEAM
NC, NS, L = 2, 16, 16
NW = NC * NS
RPW = ROWS // NW
HALF = V // 2
NV = HALF // L
CAPL = 48
CAP = CAPL * L
NEG = float("-inf")


GV = 25
GE = GV * L
NG = NV // GV


def _sc_body(lp_hbm, vals_hbm, idx_hbm, buf0, buf1, cv, ci, gmax, sem0, sem1):
    wid = lax.axis_index("s") * NC + lax.axis_index("c")
    neg16 = jnp.full((L,), NEG, jnp.float32)
    lane = lax.iota(jnp.int32, L)

    def row_body(j, _):
        r = wid * RPW + j
        base = pl.multiple_of(r * V, 8)
        cp0 = pltpu.async_copy(lp_hbm.at[pl.ds(base, HALF)], buf0, sem0)
        cp1 = pltpu.async_copy(lp_hbm.at[pl.ds(base + HALF, HALF)], buf1, sem1)

        def passa(buf, gbase):
            def b(gi, mv):
                gacc = buf[pl.ds(gi * GE, L)]
                for u in range(1, GV):
                    gacc = jnp.maximum(gacc, buf[pl.ds(gi * GE + u * L, L)])
                gmax[pl.ds((gbase + gi) * L, L)] = gacc
                return jnp.maximum(mv, gacc)
            return b

        cp0.wait()
        mv = lax.fori_loop(0, NG, passa(buf0, 0), neg16)
        cp1.wait()
        mv = lax.fori_loop(0, NG, passa(buf1, NG), mv)
        t = mv[0]
        for q in range(1, L):
            t = jnp.minimum(t, mv[q])
        tvec = jnp.full((L,), t, jnp.float32)

        def clr(i, _):
            cv[pl.ds(i * L, L)] = neg16
            ci[pl.ds(i * L, L)] = jnp.zeros((L,), jnp.int32)
            return 0

        lax.fori_loop(0, CAPL, clr, 0)

        def passb(buf, gbase, off):
            def b(gi, kl):
                gm = gmax[pl.ds((gbase + gi) * L, L)]
                hot = jnp.any(gm >= tvec)

                def scan(kl):
                    for u in range(GV):
                        v = buf[pl.ds(gi * GE + u * L, L)]
                        m = v >= tvec
                        iv = jnp.full((L,), off + u * L + gi * GE, jnp.int32) + lane
                        pos = kl * L + lane
                        plsc.store_scatter(cv, [pos], v, mask=m)
                        plsc.store_scatter(ci, [pos], iv, mask=m)
                        kl = jnp.minimum(kl + m.astype(jnp.int32), CAPL - 1)
                    return kl

                return lax.cond(hot, scan, lambda kl: kl, kl)
            return b

        kl = lax.fori_loop(0, NG, passb(buf0, 0, 0), jnp.zeros((L,), jnp.int32))
        lax.fori_loop(0, NG, passb(buf1, NG, HALF), kl)

        ob = pl.multiple_of(r * CAP, 8)
        pltpu.sync_copy(cv.at[pl.ds(0, CAP)], vals_hbm.at[pl.ds(ob, CAP)])
        pltpu.sync_copy(ci.at[pl.ds(0, CAP)], idx_hbm.at[pl.ds(ob, CAP)])
        return 0

    lax.fori_loop(0, RPW, row_body, 0)


def _sc_candidates(lp_flat):
    mesh = plsc.VectorSubcoreMesh(
        core_axis_name="c", subcore_axis_name="s", num_cores=NC, num_subcores=NS
    )
    k = pl.kernel(
        _sc_body,
        out_type=[
            jax.ShapeDtypeStruct((ROWS * CAP,), jnp.float32),
            jax.ShapeDtypeStruct((ROWS * CAP,), jnp.int32),
        ],
        mesh=mesh,
        scratch_types=[
            pltpu.VMEM((HALF,), jnp.float32),
            pltpu.VMEM((HALF,), jnp.float32),
            pltpu.VMEM((CAP,), jnp.float32),
            pltpu.VMEM((CAP,), jnp.int32),
            pltpu.VMEM((2 * NG * L,), jnp.float32),
            pltpu.SemaphoreType.DMA,
            pltpu.SemaphoreType.DMA,
        ],
        compiler_params=pltpu.CompilerParams(needs_layout_passes=False),
    )
    return k(lp_flat)


def _fix_body(cvals_ref, cidx_ref, bias_ref, mask_ref, so_ref, io_ref, bo_ref):
    BIG = jnp.int32(1 << 30)
    pen_toks = []
    scols = [None] * BEAM
    icols = [None] * BEAM
    bcols = [None] * BEAM
    for g in range(G):
        v0 = cvals_ref[:, g * CAP:(g + 1) * CAP] + bias_ref[:, g:g + 1]
        v1 = cvals_ref[:, (g + 4) * CAP:(g + 5) * CAP] + bias_ref[:, g + 4:g + 5]
        i0 = cidx_ref[:, g * CAP:(g + 1) * CAP]
        i1 = cidx_ref[:, (g + 4) * CAP:(g + 5) * CAP]
        if g > 0:
            p0 = jnp.zeros_like(v0)
            p1 = jnp.zeros_like(v1)
            for tk in pen_toks:
                p0 += (i0 == tk).astype(jnp.float32)
                p1 += (i1 == tk).astype(jnp.float32)
            v0 = v0 + DIV * p0
            v1 = v1 + DIV * p1
        v = jnp.concatenate([v0, v1], axis=1)
        f = jnp.concatenate([i0, i1 + V], axis=1)
        for k in range(2):
            mx = jnp.max(v, axis=1, keepdims=True)
            fi = jnp.min(jnp.where(v == mx, f, BIG), axis=1, keepdims=True)
            bm = (fi >= V).astype(jnp.int32)
            tok = fi - bm * V
            msk = jnp.where(bm == 0, mask_ref[:, g:g + 1], mask_ref[:, g + 4:g + 5])
            tokm = jnp.where(msk == 0, PAD, tok)
            scols[k * 4 + g] = mx
            icols[k * 4 + g] = tokm
            bcols[k * 4 + g] = bm * G + g
            pen_toks.append(tokm)
            if k == 0:
                v = jnp.where(f == fi, NEG, v)
    so_ref[...] = jnp.concatenate(scols, axis=1)
    io_ref[...] = jnp.concatenate(icols, axis=1)
    bo_ref[...] = jnp.concatenate(bcols, axis=1)


def _fixup(cvals, cidx, bias, mask, interpret=False):
    return pl.pallas_call(
        _fix_body,
        out_shape=[
            jax.ShapeDtypeStruct((BSZ, BEAM), jnp.float32),
            jax.ShapeDtypeStruct((BSZ, BEAM), jnp.int32),
            jax.ShapeDtypeStruct((BSZ, BEAM), jnp.int32),
        ],
        interpret=interpret,
    )(cvals, cidx, bias, mask)


def kernel(step, lprobs, mask_stop_search, scores, prev_indices, original_batch_idxs):
    lp_flat = lprobs.reshape(ROWS * V)
    cand_vals, cand_idx = _sc_candidates(lp_flat)
    step_i = jnp.asarray(step, jnp.int32)
    sc_step = lax.dynamic_index_in_dim(scores, step_i, axis=2, keepdims=False)
    m0 = (jnp.arange(BEAM, dtype=jnp.int32) // 4) == 0
    bias = jnp.where(step_i == 0, jnp.where(m0[None, :], 0.0, NEG), sc_step)
    out = _fixup(
        cand_vals.reshape(BSZ, BEAM * CAP),
        cand_idx.reshape(BSZ, BEAM * CAP),
        bias.astype(jnp.float32),
        mask_stop_search,
    )
    return (out[0], out[1], out[2])
